# baseline (speedup 1.0000x reference)
import functools

import jax
import jax.numpy as jnp
from jax import lax
from jax.experimental import pallas as pl
from jax.experimental.pallas import tpu as pltpu
from jax.experimental.pallas import tpu_sc as plsc

_NC = 2
_NS = 16
_NW = _NC * _NS
_CH = 100


def _elu(x):
    return jnp.where(x > 0, x, jnp.exp(x) - 1.0)


def _dotT(x, w):
    return lax.dot_general(x, w, (((1,), (1,)), ((), ())),
                           preferred_element_type=jnp.float32)


def _proj_body(n_ref, ws_ref, wr_ref, s_ref, r_ref):
    n = n_ref[...]
    s_ref[...] = _dotT(n, ws_ref[...])
    r_ref[...] = _dotT(n, wr_ref[...])


def _project_nodes(nodes2d, w1s, w1r, row_blk):
    n, d = nodes2d.shape
    return pl.pallas_call(
        _proj_body,
        grid=(n // row_blk,),
        in_specs=[pl.BlockSpec((row_blk, d), lambda i: (i, 0)),
                  pl.BlockSpec(w1s.shape, lambda i: (0, 0)),
                  pl.BlockSpec(w1r.shape, lambda i: (0, 0))],
        out_specs=[pl.BlockSpec((row_blk, d), lambda i: (i, 0)),
                   pl.BlockSpec((row_blk, d), lambda i: (i, 0))],
        out_shape=[jax.ShapeDtypeStruct((n, d), jnp.float32),
                   jax.ShapeDtypeStruct((n, d), jnp.float32)],
    )(nodes2d, w1s, w1r)


def _pair_split_body(p_ref, p0_ref, p1_ref):
    pr = p_ref[...]
    p0_ref[...] = pr[:, 0]
    p1_ref[...] = pr[:, 1]


def _pair_split(pair, row_blk):
    n = pair.shape[0]
    return pl.pallas_call(
        _pair_split_body,
        grid=(n // row_blk,),
        in_specs=[pl.BlockSpec((row_blk, 2), lambda i: (i, 0))],
        out_specs=[pl.BlockSpec((row_blk,), lambda i: (i,)),
                   pl.BlockSpec((row_blk,), lambda i: (i,))],
        out_shape=[jax.ShapeDtypeStruct((n,), jnp.int32),
                   jax.ShapeDtypeStruct((n,), jnp.int32)],
    )(pair)


def _edge_mlp_body(h_ref, e_ref, w1e_ref, b1_ref, w2_ref, b2_ref,
                   x_ref, st_ref):
    i = pl.program_id(0)
    h = h_ref[...] + _dotT(e_ref[...], w1e_ref[...]) + b1_ref[...]
    h = _elu(h)
    x = _elu(_dotT(h, w2_ref[...]) + b2_ref[...])
    c = x.shape[1]
    r = x.shape[0]
    x_ref[...] = jnp.concatenate(
        [x, jnp.ones((r, 1), jnp.float32),
         jnp.zeros((r, 127 - c), jnp.float32)], axis=1)
    ps = jnp.sum(x, axis=0, keepdims=True)
    pq = jnp.sum(x * x, axis=0, keepdims=True)
    upd = jnp.concatenate([ps, pq, jnp.zeros((6, c), jnp.float32)], axis=0)

    @pl.when(i == 0)
    def _():
        st_ref[...] = upd

    @pl.when(i > 0)
    def _():
        st_ref[...] += upd


def _edge_mlp(h, e, w1e, b1, w2, b2, row_blk):
    n, dh = h.shape
    de = e.shape[1]
    c = w2.shape[0]
    return pl.pallas_call(
        _edge_mlp_body,
        grid=(n // row_blk,),
        in_specs=[pl.BlockSpec((row_blk, dh), lambda i: (i, 0)),
                  pl.BlockSpec((row_blk, de), lambda i: (i, 0)),
                  pl.BlockSpec(w1e.shape, lambda i: (0, 0)),
                  pl.BlockSpec((1, dh), lambda i: (0, 0)),
                  pl.BlockSpec(w2.shape, lambda i: (0, 0)),
                  pl.BlockSpec((1, c), lambda i: (0, 0))],
        out_specs=[pl.BlockSpec((row_blk, 128), lambda i: (i, 0)),
                   pl.BlockSpec((8, c), lambda i: (0, 0))],
        out_shape=[jax.ShapeDtypeStruct((n, 128), jnp.float32),
                   jax.ShapeDtypeStruct((8, c), jnp.float32)],
    )(h, e, w1e, b1, w2, b2)


def _edge_fin_body(x_ref, e_ref, a_ref, c_ref, ne_ref):
    dc = e_ref.shape[1]
    y = x_ref[:, :dc] * a_ref[...] + c_ref[...]
    ne_ref[...] = y + e_ref[...]


def _edge_fin(x128, e, a, c, row_blk):
    n, dc = e.shape
    return pl.pallas_call(
        _edge_fin_body,
        grid=(n // row_blk,),
        in_specs=[pl.BlockSpec((row_blk, 128), lambda i: (i, 0)),
                  pl.BlockSpec((row_blk, dc), lambda i: (i, 0)),
                  pl.BlockSpec((1, dc), lambda i: (0, 0)),
                  pl.BlockSpec((1, dc), lambda i: (0, 0))],
        out_specs=pl.BlockSpec((row_blk, dc), lambda i: (i, 0)),
        out_shape=jax.ShapeDtypeStruct((n, dc), jnp.float32),
    )(x128, e, a, c)


def _node_mlp_body(n_ref, s0_ref, s1_ref, ae_ref, ce_ref, w1a_ref, w1b_ref,
                   b1_ref, w2_ref, b2_ref, x_ref, st_ref):
    i = pl.program_id(0)
    dc = w1b_ref.shape[1]
    st = s0_ref[...] + s1_ref[...]
    nf = st[:, :dc] * ae_ref[...] + st[:, dc:dc + 1] * ce_ref[...]
    h = (_dotT(n_ref[...], w1a_ref[...]) + _dotT(nf, w1b_ref[...])
         + b1_ref[...])
    h = _elu(h)
    x = _elu(_dotT(h, w2_ref[...]) + b2_ref[...])
    x_ref[...] = x
    c = x.shape[1]
    ps = jnp.sum(x, axis=0, keepdims=True)
    pq = jnp.sum(x * x, axis=0, keepdims=True)
    upd = jnp.concatenate([ps, pq, jnp.zeros((6, c), jnp.float32)], axis=0)

    @pl.when(i == 0)
    def _():
        st_ref[...] = upd

    @pl.when(i > 0)
    def _():
        st_ref[...] += upd


def _node_mlp(n2, s0, s1, ae, ce, w1a, w1b, b1, w2, b2, row_blk):
    n, dn = n2.shape
    dc = s0.shape[1]
    dce = ae.shape[1]
    dh = w1a.shape[0]
    do = w2.shape[0]
    return pl.pallas_call(
        _node_mlp_body,
        grid=(n // row_blk,),
        in_specs=[pl.BlockSpec((row_blk, dn), lambda i: (i, 0)),
                  pl.BlockSpec((row_blk, dc), lambda i: (i, 0)),
                  pl.BlockSpec((row_blk, dc), lambda i: (i, 0)),
                  pl.BlockSpec((1, dce), lambda i: (0, 0)),
                  pl.BlockSpec((1, dce), lambda i: (0, 0)),
                  pl.BlockSpec(w1a.shape, lambda i: (0, 0)),
                  pl.BlockSpec(w1b.shape, lambda i: (0, 0)),
                  pl.BlockSpec((1, dh), lambda i: (0, 0)),
                  pl.BlockSpec(w2.shape, lambda i: (0, 0)),
                  pl.BlockSpec((1, do), lambda i: (0, 0))],
        out_specs=[pl.BlockSpec((row_blk, do), lambda i: (i, 0)),
                   pl.BlockSpec((8, do), lambda i: (0, 0))],
        out_shape=[jax.ShapeDtypeStruct((n, do), jnp.float32),
                   jax.ShapeDtypeStruct((8, do), jnp.float32)],
    )(n2, s0, s1, ae, ce, w1a, w1b, b1, w2, b2)


def _node_fin_body(x_ref, n_ref, a_ref, c_ref, o_ref):
    o_ref[...] = x_ref[...] * a_ref[...] + c_ref[...] + n_ref[...]


def _node_fin(x, n2, a, c, row_blk):
    n, dn = x.shape
    return pl.pallas_call(
        _node_fin_body,
        grid=(n // row_blk,),
        in_specs=[pl.BlockSpec((row_blk, dn), lambda i: (i, 0)),
                  pl.BlockSpec((row_blk, dn), lambda i: (i, 0)),
                  pl.BlockSpec((1, dn), lambda i: (0, 0)),
                  pl.BlockSpec((1, dn), lambda i: (0, 0))],
        out_specs=pl.BlockSpec((row_blk, dn), lambda i: (i, 0)),
        out_shape=jax.ShapeDtypeStruct((n, dn), jnp.float32),
    )(x, n2, a, c)


_CHF = 128


def _sc_gather_sum(e_total, d):
    epw = e_total // _NW
    ncf = epw // _CHF
    tail = epw - ncf * _CHF
    assert ncf % 2 == 0 and e_total % _NW == 0 and tail % 8 == 0
    mesh = plsc.VectorSubcoreMesh(core_axis_name="c", subcore_axis_name="s")

    @functools.partial(
        pl.kernel,
        out_type=jax.ShapeDtypeStruct((e_total, d), jnp.float32),
        mesh=mesh,
        scratch_types=[
            pltpu.VMEM((2, _CHF), jnp.int32),
            pltpu.VMEM((2, _CHF), jnp.int32),
            pltpu.VMEM((2, _CHF, d), jnp.float32),
            pltpu.VMEM((2, _CHF, d), jnp.float32),
            pltpu.VMEM((2, _CHF, d), jnp.float32),
            pltpu.SemaphoreType.DMA((2,)),
            pltpu.SemaphoreType.DMA((2,)),
            pltpu.SemaphoreType.DMA((2,)),
            pltpu.SemaphoreType.DMA((2,)),
        ],
    )
    def k(ts_hbm, tr_hbm, i0_hbm, i1_hbm, out_hbm, ia, ib, bufa, bufb, bufo,
          si, sa, sb, so):
        wid = lax.axis_index("s") * _NC + lax.axis_index("c")
        base = wid * epw

        def idx_issue(j, s):
            pltpu.async_copy(i0_hbm.at[pl.ds(base + j * _CHF, _CHF)],
                             ia.at[s], si.at[s])
            pltpu.async_copy(i1_hbm.at[pl.ds(base + j * _CHF, _CHF)],
                             ib.at[s], si.at[s])

        def idx_wait(j, s):
            pltpu.make_async_copy(i0_hbm.at[pl.ds(base + j * _CHF, _CHF)],
                                  ia.at[s], si.at[s]).wait()
            pltpu.make_async_copy(i1_hbm.at[pl.ds(base + j * _CHF, _CHF)],
                                  ib.at[s], si.at[s]).wait()

        def g_issue(s):
            pltpu.async_copy(ts_hbm.at[ia.at[s]], bufa.at[s], sa.at[s])
            pltpu.async_copy(tr_hbm.at[ib.at[s]], bufb.at[s], sb.at[s])

        def g_wait(s):
            pltpu.make_async_copy(ts_hbm.at[ia.at[s]], bufa.at[s],
                                  sa.at[s]).wait()
            pltpu.make_async_copy(tr_hbm.at[ib.at[s]], bufb.at[s],
                                  sb.at[s]).wait()

        def out_wait(j, s):
            pltpu.make_async_copy(bufo.at[s],
                                  out_hbm.at[pl.ds(base + j * _CHF, _CHF)],
                                  so.at[s]).wait()

        def add_rows(s, nrows):
            def row(r, cc):
                def col(q, c3):
                    sl = (s, r, pl.ds(q * 16, 16))
                    bufo[sl] = bufa[sl] + bufb[sl]
                    return c3
                return lax.fori_loop(0, d // 16, col, cc, unroll=True)
            lax.fori_loop(0, nrows, row, 0)

        idx_issue(0, 0)
        idx_issue(1, 1)
        idx_wait(0, 0)
        g_issue(0)

        def step(s, jj):
            g_wait(s)

            @pl.when(jj >= 2)
            def _():
                out_wait(jj - 2, s)

            add_rows(s, _CHF)
            pltpu.async_copy(bufo.at[s],
                             out_hbm.at[pl.ds(base + jj * _CHF, _CHF)],
                             so.at[s])

            @pl.when(jj + 1 < ncf)
            def _():
                idx_wait(jj + 1, 1 - s)
                g_issue(1 - s)

            @pl.when(jj + 2 < ncf)
            def _():
                idx_issue(jj + 2, s)

        def body(j2, carry):
            step(0, 2 * j2)
            step(1, 2 * j2 + 1)
            return carry

        lax.fori_loop(0, ncf // 2, body, 0)
        out_wait(ncf - 2, 0)
        out_wait(ncf - 1, 1)

        if tail:
            toff = base + ncf * _CHF
            pltpu.sync_copy(i0_hbm.at[pl.ds(toff, tail)],
                            ia.at[0, pl.ds(0, tail)])
            pltpu.sync_copy(i1_hbm.at[pl.ds(toff, tail)],
                            ib.at[0, pl.ds(0, tail)])
            ca = pltpu.async_copy(ts_hbm.at[ia.at[0, pl.ds(0, tail)]],
                                  bufa.at[0, pl.ds(0, tail)], sa.at[0])
            cb = pltpu.async_copy(tr_hbm.at[ib.at[0, pl.ds(0, tail)]],
                                  bufb.at[0, pl.ds(0, tail)], sb.at[0])
            ca.wait()
            cb.wait()
            add_rows(0, tail)
            pltpu.sync_copy(bufo.at[0, pl.ds(0, tail)],
                            out_hbm.at[pl.ds(toff, tail)])

    return k


def _sc_scatter_add(e_total, n_nodes, d):
    epw = e_total // _NW
    ncf = epw // _CHF
    tail = epw - ncf * _CHF
    assert ncf % 2 == 0 and e_total % _NW == 0 and tail % 8 == 0
    npad = ((n_nodes + _NS * _CHF - 1) // (_NS * _CHF)) * _NS * _CHF
    blk_per_tile = npad // (_NS * _CHF)
    mesh = plsc.VectorSubcoreMesh(core_axis_name="c", subcore_axis_name="s")

    @functools.partial(
        pl.kernel,
        out_type=jax.ShapeDtypeStruct((2, n_nodes, d), jnp.float32),
        mesh=mesh,
        scratch_types=[
            pltpu.VMEM((2, _CHF), jnp.int32),
            pltpu.VMEM((2, _CHF), jnp.int32),
            pltpu.VMEM((2, _CHF, d), jnp.float32),
            pltpu.VMEM_SHARED((npad, d), jnp.float32),
            pltpu.SemaphoreType.DMA((2,)),
            pltpu.SemaphoreType.DMA((2,)),
            pltpu.SemaphoreType.DMA((2,)),
        ],
    )
    def k(y_hbm, p0_hbm, p1_hbm, out_hbm, ia, ib, ybuf, acc,
          sl, t0, t1):
        cid = lax.axis_index("c")
        sid = lax.axis_index("s")

        def zrow(r, cc):
            def zcol(q, c3):
                ybuf[0, r, pl.ds(q * 16, 16)] = jnp.zeros((16,), jnp.float32)
                return c3
            return lax.fori_loop(0, d // 16, zcol, cc, unroll=True)
        lax.fori_loop(0, _CHF, zrow, 0)
        for kk in range(blk_per_tile):
            pltpu.sync_copy(ybuf.at[0],
                            acc.at[pl.ds((sid * blk_per_tile + kk) * _CHF,
                                         _CHF)])

        wid = sid * _NC + cid
        base = wid * epw
        plsc.subcore_barrier()

        def load(j, s):
            pltpu.async_copy(p0_hbm.at[pl.ds(base + j * _CHF, _CHF)],
                             ia.at[s], sl.at[s])
            pltpu.async_copy(p1_hbm.at[pl.ds(base + j * _CHF, _CHF)],
                             ib.at[s], sl.at[s])
            pltpu.async_copy(y_hbm.at[pl.ds(base + j * _CHF, _CHF)],
                             ybuf.at[s], sl.at[s])

        def load_wait(j, s):
            pltpu.make_async_copy(p0_hbm.at[pl.ds(base + j * _CHF, _CHF)],
                                  ia.at[s], sl.at[s]).wait()
            pltpu.make_async_copy(p1_hbm.at[pl.ds(base + j * _CHF, _CHF)],
                                  ib.at[s], sl.at[s]).wait()
            pltpu.make_async_copy(y_hbm.at[pl.ds(base + j * _CHF, _CHF)],
                                  ybuf.at[s], sl.at[s]).wait()

        load(0, 0)
        load(1, 1)

        def step(s, jj):
            load_wait(jj, s)
            c0 = pltpu.async_copy(ybuf.at[s], acc.at[ia.at[s]], t0.at[s],
                                  add=True)
            c1 = pltpu.async_copy(ybuf.at[s], acc.at[ib.at[s]], t1.at[s],
                                  add=True)
            c0.wait()
            c1.wait()

            @pl.when(jj + 2 < ncf)
            def _():
                load(jj + 2, s)

        def body(j2, carry):
            step(0, 2 * j2)
            step(1, 2 * j2 + 1)
            return carry

        lax.fori_loop(0, ncf // 2, body, 0)

        if tail:
            toff = base + ncf * _CHF
            pltpu.sync_copy(p0_hbm.at[pl.ds(toff, tail)],
                            ia.at[0, pl.ds(0, tail)])
            pltpu.sync_copy(p1_hbm.at[pl.ds(toff, tail)],
                            ib.at[0, pl.ds(0, tail)])
            pltpu.sync_copy(y_hbm.at[pl.ds(toff, tail)],
                            ybuf.at[0, pl.ds(0, tail)])
            c0 = pltpu.async_copy(ybuf.at[0, pl.ds(0, tail)],
                                  acc.at[ia.at[0, pl.ds(0, tail)]],
                                  t0.at[0], add=True)
            c1 = pltpu.async_copy(ybuf.at[0, pl.ds(0, tail)],
                                  acc.at[ib.at[0, pl.ds(0, tail)]],
                                  t1.at[0], add=True)
            c0.wait()
            c1.wait()

        plsc.subcore_barrier()

        @pl.when(sid == 0)
        def _():
            pltpu.sync_copy(acc.at[pl.ds(0, n_nodes)], out_hbm.at[cid])

    return k


def kernel(nodes, edges, pair, eW1, eb1, eW2, eb2, eg, ebt,
           nW1, nb1, nW2, nb2, ng, nbt):
    n_nodes = nodes.shape[1]
    n_edges = edges.shape[1]
    dn = nodes.shape[2]
    de = edges.shape[2]
    n2 = nodes[0]
    e2 = edges[0]

    w1s = eW1[:, :dn]
    w1r = eW1[:, dn:2 * dn]
    w1e = eW1[:, 2 * dn:]

    ts, tr = _project_nodes(n2, w1s, w1r, 1000)

    p0, p1 = _pair_split(pair, 512)

    h = _sc_gather_sum(n_edges, dn)(ts, tr, p0, p1)

    x128, est = _edge_mlp(h, e2, w1e, eb1.reshape(1, -1), eW2,
                          eb2.reshape(1, -1), 512)
    mean_e = est[0] / n_edges
    var_e = est[1] / n_edges - mean_e * mean_e
    a_e = eg * lax.rsqrt(var_e + 1e-5)
    c_e = ebt - mean_e * a_e

    s_part = _sc_scatter_add(n_edges, n_nodes, 128)(x128, p0, p1)

    new_e = _edge_fin(x128, e2, a_e.reshape(1, -1), c_e.reshape(1, -1), 512)

    x_n, nst = _node_mlp(n2, s_part[0], s_part[1], a_e.reshape(1, -1),
                         c_e.reshape(1, -1), nW1[:, :dn],
                         nW1[:, dn:], nb1.reshape(1, -1), nW2,
                         nb2.reshape(1, -1), 1000)
    mean_n = nst[0] / n_nodes
    var_n = nst[1] / n_nodes - mean_n * mean_n
    a_n = ng * lax.rsqrt(var_n + 1e-5)
    c_n = nbt - mean_n * a_n

    new_n = _node_fin(x_n, n2, a_n.reshape(1, -1), c_n.reshape(1, -1), 1000)

    return new_n.reshape(1, n_nodes, dn), new_e.reshape(1, n_edges, de)


# speedup vs baseline: 1.2192x; 1.2192x over previous
import functools

import jax
import jax.numpy as jnp
from jax import lax
from jax.experimental import pallas as pl
from jax.experimental.pallas import tpu as pltpu
from jax.experimental.pallas import tpu_sc as plsc

_NC = 2
_NS = 16
_NW = _NC * _NS
_CH = 100


def _elu(x):
    return jnp.where(x > 0, x, jnp.exp(x) - 1.0)


def _dotT(x, w):
    return lax.dot_general(x, w, (((1,), (1,)), ((), ())),
                           preferred_element_type=jnp.float32)


def _proj_body(n_ref, ws_ref, wr_ref, s_ref, r_ref):
    n = n_ref[...]
    s_ref[...] = _dotT(n, ws_ref[...])
    r_ref[...] = _dotT(n, wr_ref[...])


def _project_nodes(nodes2d, w1s, w1r, row_blk):
    n, d = nodes2d.shape
    return pl.pallas_call(
        _proj_body,
        grid=(n // row_blk,),
        in_specs=[pl.BlockSpec((row_blk, d), lambda i: (i, 0)),
                  pl.BlockSpec(w1s.shape, lambda i: (0, 0)),
                  pl.BlockSpec(w1r.shape, lambda i: (0, 0))],
        out_specs=[pl.BlockSpec((row_blk, d), lambda i: (i, 0)),
                   pl.BlockSpec((row_blk, d), lambda i: (i, 0))],
        out_shape=[jax.ShapeDtypeStruct((n, d), jnp.float32),
                   jax.ShapeDtypeStruct((n, d), jnp.float32)],
    )(nodes2d, w1s, w1r)


def _pair_split_body(p_ref, p0_ref, p1_ref):
    pr = p_ref[...]
    p0_ref[...] = pr[0]
    p1_ref[...] = pr[1]


def _pair_split(pair_t, row_blk):
    n = pair_t.shape[1]
    return pl.pallas_call(
        _pair_split_body,
        grid=(n // row_blk,),
        in_specs=[pl.BlockSpec((2, row_blk), lambda i: (0, i))],
        out_specs=[pl.BlockSpec((row_blk,), lambda i: (i,)),
                   pl.BlockSpec((row_blk,), lambda i: (i,))],
        out_shape=[jax.ShapeDtypeStruct((n,), jnp.int32),
                   jax.ShapeDtypeStruct((n,), jnp.int32)],
    )(pair_t)


def _edge_mlp_body(h_ref, et_ref, w1e_ref, b1_ref, w2_ref, b2_ref,
                   x_ref, st_ref):
    i = pl.program_id(0)
    ew = lax.dot_general(et_ref[...], w1e_ref[...], (((0,), (1,)), ((), ())),
                         preferred_element_type=jnp.float32)
    h = h_ref[...] + ew + b1_ref[...]
    h = _elu(h)
    x = _elu(_dotT(h, w2_ref[...]) + b2_ref[...])
    c = x.shape[1]
    r = x.shape[0]
    x_ref[...] = jnp.concatenate(
        [x, jnp.ones((r, 1), jnp.float32),
         jnp.zeros((r, 127 - c), jnp.float32)], axis=1)
    ps = jnp.sum(x, axis=0, keepdims=True)
    pq = jnp.sum(x * x, axis=0, keepdims=True)
    upd = jnp.concatenate([ps, pq, jnp.zeros((6, c), jnp.float32)], axis=0)

    @pl.when(i == 0)
    def _():
        st_ref[...] = upd

    @pl.when(i > 0)
    def _():
        st_ref[...] += upd


def _edge_mlp(h, e_t, w1e, b1, w2, b2, row_blk):
    n, dh = h.shape
    de = e_t.shape[0]
    c = w2.shape[0]
    return pl.pallas_call(
        _edge_mlp_body,
        grid=(n // row_blk,),
        in_specs=[pl.BlockSpec((row_blk, dh), lambda i: (i, 0)),
                  pl.BlockSpec((de, row_blk), lambda i: (0, i)),
                  pl.BlockSpec(w1e.shape, lambda i: (0, 0)),
                  pl.BlockSpec((1, dh), lambda i: (0, 0)),
                  pl.BlockSpec(w2.shape, lambda i: (0, 0)),
                  pl.BlockSpec((1, c), lambda i: (0, 0))],
        out_specs=[pl.BlockSpec((row_blk, 128), lambda i: (i, 0)),
                   pl.BlockSpec((8, c), lambda i: (0, 0))],
        out_shape=[jax.ShapeDtypeStruct((n, 128), jnp.float32),
                   jax.ShapeDtypeStruct((8, c), jnp.float32)],
    )(h, e_t, w1e, b1, w2, b2)


def _edge_fin_body(x_ref, et_ref, a_ref, c_ref, net_ref):
    dc = et_ref.shape[0]
    y = x_ref[:, :dc] * a_ref[...] + c_ref[...]
    net_ref[...] = jnp.transpose(y) + et_ref[...]


def _edge_fin(x128, e_t, a, c, row_blk):
    dc, n = e_t.shape
    return pl.pallas_call(
        _edge_fin_body,
        grid=(n // row_blk,),
        in_specs=[pl.BlockSpec((row_blk, 128), lambda i: (i, 0)),
                  pl.BlockSpec((dc, row_blk), lambda i: (0, i)),
                  pl.BlockSpec((1, dc), lambda i: (0, 0)),
                  pl.BlockSpec((1, dc), lambda i: (0, 0))],
        out_specs=pl.BlockSpec((dc, row_blk), lambda i: (0, i)),
        out_shape=jax.ShapeDtypeStruct((dc, n), jnp.float32),
    )(x128, e_t, a, c)


def _node_mlp_body(n_ref, s0_ref, s1_ref, ae_ref, ce_ref, w1a_ref, w1b_ref,
                   b1_ref, w2_ref, b2_ref, x_ref, st_ref):
    i = pl.program_id(0)
    dc = w1b_ref.shape[1]
    st = s0_ref[...] + s1_ref[...]
    nf = st[:, :dc] * ae_ref[...] + st[:, dc:dc + 1] * ce_ref[...]
    h = (_dotT(n_ref[...], w1a_ref[...]) + _dotT(nf, w1b_ref[...])
         + b1_ref[...])
    h = _elu(h)
    x = _elu(_dotT(h, w2_ref[...]) + b2_ref[...])
    x_ref[...] = x
    c = x.shape[1]
    ps = jnp.sum(x, axis=0, keepdims=True)
    pq = jnp.sum(x * x, axis=0, keepdims=True)
    upd = jnp.concatenate([ps, pq, jnp.zeros((6, c), jnp.float32)], axis=0)

    @pl.when(i == 0)
    def _():
        st_ref[...] = upd

    @pl.when(i > 0)
    def _():
        st_ref[...] += upd


def _node_mlp(n2, s0, s1, ae, ce, w1a, w1b, b1, w2, b2, row_blk):
    n, dn = n2.shape
    dc = s0.shape[1]
    dce = ae.shape[1]
    dh = w1a.shape[0]
    do = w2.shape[0]
    return pl.pallas_call(
        _node_mlp_body,
        grid=(n // row_blk,),
        in_specs=[pl.BlockSpec((row_blk, dn), lambda i: (i, 0)),
                  pl.BlockSpec((row_blk, dc), lambda i: (i, 0)),
                  pl.BlockSpec((row_blk, dc), lambda i: (i, 0)),
                  pl.BlockSpec((1, dce), lambda i: (0, 0)),
                  pl.BlockSpec((1, dce), lambda i: (0, 0)),
                  pl.BlockSpec(w1a.shape, lambda i: (0, 0)),
                  pl.BlockSpec(w1b.shape, lambda i: (0, 0)),
                  pl.BlockSpec((1, dh), lambda i: (0, 0)),
                  pl.BlockSpec(w2.shape, lambda i: (0, 0)),
                  pl.BlockSpec((1, do), lambda i: (0, 0))],
        out_specs=[pl.BlockSpec((row_blk, do), lambda i: (i, 0)),
                   pl.BlockSpec((8, do), lambda i: (0, 0))],
        out_shape=[jax.ShapeDtypeStruct((n, do), jnp.float32),
                   jax.ShapeDtypeStruct((8, do), jnp.float32)],
    )(n2, s0, s1, ae, ce, w1a, w1b, b1, w2, b2)


def _node_fin_body(x_ref, n_ref, a_ref, c_ref, o_ref):
    o_ref[...] = x_ref[...] * a_ref[...] + c_ref[...] + n_ref[...]


def _node_fin(x, n2, a, c, row_blk):
    n, dn = x.shape
    return pl.pallas_call(
        _node_fin_body,
        grid=(n // row_blk,),
        in_specs=[pl.BlockSpec((row_blk, dn), lambda i: (i, 0)),
                  pl.BlockSpec((row_blk, dn), lambda i: (i, 0)),
                  pl.BlockSpec((1, dn), lambda i: (0, 0)),
                  pl.BlockSpec((1, dn), lambda i: (0, 0))],
        out_specs=pl.BlockSpec((row_blk, dn), lambda i: (i, 0)),
        out_shape=jax.ShapeDtypeStruct((n, dn), jnp.float32),
    )(x, n2, a, c)


_CHF = 128


def _sc_gather_sum(e_total, d):
    epw = e_total // _NW
    ncf = epw // _CHF
    tail = epw - ncf * _CHF
    assert ncf % 2 == 0 and e_total % _NW == 0 and tail % 8 == 0
    mesh = plsc.VectorSubcoreMesh(core_axis_name="c", subcore_axis_name="s")

    @functools.partial(
        pl.kernel,
        out_type=jax.ShapeDtypeStruct((e_total, d), jnp.float32),
        mesh=mesh,
        scratch_types=[
            pltpu.VMEM((2, _CHF), jnp.int32),
            pltpu.VMEM((2, _CHF), jnp.int32),
            pltpu.VMEM((2, _CHF, d), jnp.float32),
            pltpu.VMEM((2, _CHF, d), jnp.float32),
            pltpu.VMEM((2, _CHF, d), jnp.float32),
            pltpu.SemaphoreType.DMA((2,)),
            pltpu.SemaphoreType.DMA((2,)),
            pltpu.SemaphoreType.DMA((2,)),
            pltpu.SemaphoreType.DMA((2,)),
        ],
    )
    def k(ts_hbm, tr_hbm, i0_hbm, i1_hbm, out_hbm, ia, ib, bufa, bufb, bufo,
          si, sa, sb, so):
        wid = lax.axis_index("s") * _NC + lax.axis_index("c")
        base = wid * epw

        def idx_issue(j, s):
            pltpu.async_copy(i0_hbm.at[pl.ds(base + j * _CHF, _CHF)],
                             ia.at[s], si.at[s])
            pltpu.async_copy(i1_hbm.at[pl.ds(base + j * _CHF, _CHF)],
                             ib.at[s], si.at[s])

        def idx_wait(j, s):
            pltpu.make_async_copy(i0_hbm.at[pl.ds(base + j * _CHF, _CHF)],
                                  ia.at[s], si.at[s]).wait()
            pltpu.make_async_copy(i1_hbm.at[pl.ds(base + j * _CHF, _CHF)],
                                  ib.at[s], si.at[s]).wait()

        def g_issue(s):
            pltpu.async_copy(ts_hbm.at[ia.at[s]], bufa.at[s], sa.at[s])
            pltpu.async_copy(tr_hbm.at[ib.at[s]], bufb.at[s], sb.at[s])

        def g_wait(s):
            pltpu.make_async_copy(ts_hbm.at[ia.at[s]], bufa.at[s],
                                  sa.at[s]).wait()
            pltpu.make_async_copy(tr_hbm.at[ib.at[s]], bufb.at[s],
                                  sb.at[s]).wait()

        def out_wait(j, s):
            pltpu.make_async_copy(bufo.at[s],
                                  out_hbm.at[pl.ds(base + j * _CHF, _CHF)],
                                  so.at[s]).wait()

        def add_rows(s, nrows):
            def row(r, cc):
                def col(q, c3):
                    sl = (s, r, pl.ds(q * 16, 16))
                    bufo[sl] = bufa[sl] + bufb[sl]
                    return c3
                return lax.fori_loop(0, d // 16, col, cc, unroll=True)
            lax.fori_loop(0, nrows, row, 0)

        idx_issue(0, 0)
        idx_issue(1, 1)
        idx_wait(0, 0)
        g_issue(0)

        def step(s, jj):
            g_wait(s)

            @pl.when(jj >= 2)
            def _():
                out_wait(jj - 2, s)

            add_rows(s, _CHF)
            pltpu.async_copy(bufo.at[s],
                             out_hbm.at[pl.ds(base + jj * _CHF, _CHF)],
                             so.at[s])

            @pl.when(jj + 1 < ncf)
            def _():
                idx_wait(jj + 1, 1 - s)
                g_issue(1 - s)

            @pl.when(jj + 2 < ncf)
            def _():
                idx_issue(jj + 2, s)

        def body(j2, carry):
            step(0, 2 * j2)
            step(1, 2 * j2 + 1)
            return carry

        lax.fori_loop(0, ncf // 2, body, 0)
        out_wait(ncf - 2, 0)
        out_wait(ncf - 1, 1)

        if tail:
            toff = base + ncf * _CHF
            pltpu.sync_copy(i0_hbm.at[pl.ds(toff, tail)],
                            ia.at[0, pl.ds(0, tail)])
            pltpu.sync_copy(i1_hbm.at[pl.ds(toff, tail)],
                            ib.at[0, pl.ds(0, tail)])
            ca = pltpu.async_copy(ts_hbm.at[ia.at[0, pl.ds(0, tail)]],
                                  bufa.at[0, pl.ds(0, tail)], sa.at[0])
            cb = pltpu.async_copy(tr_hbm.at[ib.at[0, pl.ds(0, tail)]],
                                  bufb.at[0, pl.ds(0, tail)], sb.at[0])
            ca.wait()
            cb.wait()
            add_rows(0, tail)
            pltpu.sync_copy(bufo.at[0, pl.ds(0, tail)],
                            out_hbm.at[pl.ds(toff, tail)])

    return k


def _sc_scatter_add(e_total, n_nodes, d):
    epw = e_total // _NW
    ncf = epw // _CHF
    tail = epw - ncf * _CHF
    assert ncf % 2 == 0 and e_total % _NW == 0 and tail % 8 == 0
    npad = ((n_nodes + _NS * _CHF - 1) // (_NS * _CHF)) * _NS * _CHF
    blk_per_tile = npad // (_NS * _CHF)
    mesh = plsc.VectorSubcoreMesh(core_axis_name="c", subcore_axis_name="s")

    @functools.partial(
        pl.kernel,
        out_type=jax.ShapeDtypeStruct((2, n_nodes, d), jnp.float32),
        mesh=mesh,
        scratch_types=[
            pltpu.VMEM((2, _CHF), jnp.int32),
            pltpu.VMEM((2, _CHF), jnp.int32),
            pltpu.VMEM((2, _CHF, d), jnp.float32),
            pltpu.VMEM_SHARED((npad, d), jnp.float32),
            pltpu.SemaphoreType.DMA((2,)),
            pltpu.SemaphoreType.DMA((2,)),
            pltpu.SemaphoreType.DMA((2,)),
        ],
    )
    def k(y_hbm, p0_hbm, p1_hbm, out_hbm, ia, ib, ybuf, acc,
          sl, t0, t1):
        cid = lax.axis_index("c")
        sid = lax.axis_index("s")

        def zrow(r, cc):
            def zcol(q, c3):
                ybuf[0, r, pl.ds(q * 16, 16)] = jnp.zeros((16,), jnp.float32)
                return c3
            return lax.fori_loop(0, d // 16, zcol, cc, unroll=True)
        lax.fori_loop(0, _CHF, zrow, 0)
        for kk in range(blk_per_tile):
            pltpu.sync_copy(ybuf.at[0],
                            acc.at[pl.ds((sid * blk_per_tile + kk) * _CHF,
                                         _CHF)])

        wid = sid * _NC + cid
        base = wid * epw
        plsc.subcore_barrier()

        def load(j, s):
            pltpu.async_copy(p0_hbm.at[pl.ds(base + j * _CHF, _CHF)],
                             ia.at[s], sl.at[s])
            pltpu.async_copy(p1_hbm.at[pl.ds(base + j * _CHF, _CHF)],
                             ib.at[s], sl.at[s])
            pltpu.async_copy(y_hbm.at[pl.ds(base + j * _CHF, _CHF)],
                             ybuf.at[s], sl.at[s])

        def load_wait(j, s):
            pltpu.make_async_copy(p0_hbm.at[pl.ds(base + j * _CHF, _CHF)],
                                  ia.at[s], sl.at[s]).wait()
            pltpu.make_async_copy(p1_hbm.at[pl.ds(base + j * _CHF, _CHF)],
                                  ib.at[s], sl.at[s]).wait()
            pltpu.make_async_copy(y_hbm.at[pl.ds(base + j * _CHF, _CHF)],
                                  ybuf.at[s], sl.at[s]).wait()

        load(0, 0)
        load(1, 1)

        def step(s, jj):
            load_wait(jj, s)
            c0 = pltpu.async_copy(ybuf.at[s], acc.at[ia.at[s]], t0.at[s],
                                  add=True)
            c1 = pltpu.async_copy(ybuf.at[s], acc.at[ib.at[s]], t1.at[s],
                                  add=True)
            c0.wait()
            c1.wait()

            @pl.when(jj + 2 < ncf)
            def _():
                load(jj + 2, s)

        def body(j2, carry):
            step(0, 2 * j2)
            step(1, 2 * j2 + 1)
            return carry

        lax.fori_loop(0, ncf // 2, body, 0)

        if tail:
            toff = base + ncf * _CHF
            pltpu.sync_copy(p0_hbm.at[pl.ds(toff, tail)],
                            ia.at[0, pl.ds(0, tail)])
            pltpu.sync_copy(p1_hbm.at[pl.ds(toff, tail)],
                            ib.at[0, pl.ds(0, tail)])
            pltpu.sync_copy(y_hbm.at[pl.ds(toff, tail)],
                            ybuf.at[0, pl.ds(0, tail)])
            c0 = pltpu.async_copy(ybuf.at[0, pl.ds(0, tail)],
                                  acc.at[ia.at[0, pl.ds(0, tail)]],
                                  t0.at[0], add=True)
            c1 = pltpu.async_copy(ybuf.at[0, pl.ds(0, tail)],
                                  acc.at[ib.at[0, pl.ds(0, tail)]],
                                  t1.at[0], add=True)
            c0.wait()
            c1.wait()

        plsc.subcore_barrier()

        @pl.when(sid == 0)
        def _():
            pltpu.sync_copy(acc.at[pl.ds(0, n_nodes)], out_hbm.at[cid])

    return k


def kernel(nodes, edges, pair, eW1, eb1, eW2, eb2, eg, ebt,
           nW1, nb1, nW2, nb2, ng, nbt):
    n_nodes = nodes.shape[1]
    n_edges = edges.shape[1]
    dn = nodes.shape[2]
    de = edges.shape[2]
    n2 = nodes[0]
    e_t = jnp.transpose(edges[0])
    pair_t = jnp.transpose(pair)

    w1s = eW1[:, :dn]
    w1r = eW1[:, dn:2 * dn]
    w1e = eW1[:, 2 * dn:]

    ts, tr = _project_nodes(n2, w1s, w1r, 1000)

    p0, p1 = _pair_split(pair_t, 512)

    h = _sc_gather_sum(n_edges, dn)(ts, tr, p0, p1)

    x128, est = _edge_mlp(h, e_t, w1e, eb1.reshape(1, -1), eW2,
                          eb2.reshape(1, -1), 512)
    mean_e = est[0] / n_edges
    var_e = est[1] / n_edges - mean_e * mean_e
    a_e = eg * lax.rsqrt(var_e + 1e-5)
    c_e = ebt - mean_e * a_e

    s_part = _sc_scatter_add(n_edges, n_nodes, 128)(x128, p0, p1)

    new_et = _edge_fin(x128, e_t, a_e.reshape(1, -1), c_e.reshape(1, -1),
                       512)
    new_e = jnp.transpose(new_et)

    x_n, nst = _node_mlp(n2, s_part[0], s_part[1], a_e.reshape(1, -1),
                         c_e.reshape(1, -1), nW1[:, :dn],
                         nW1[:, dn:], nb1.reshape(1, -1), nW2,
                         nb2.reshape(1, -1), 1000)
    mean_n = nst[0] / n_nodes
    var_n = nst[1] / n_nodes - mean_n * mean_n
    a_n = ng * lax.rsqrt(var_n + 1e-5)
    c_n = nbt - mean_n * a_n

    new_n = _node_fin(x_n, n2, a_n.reshape(1, -1), c_n.reshape(1, -1), 1000)

    return new_n.reshape(1, n_nodes, dn), new_e.reshape(1, n_edges, de)


# speedup vs baseline: 2.6763x; 2.1951x over previous
import functools

import jax
import jax.numpy as jnp
from jax import lax
from jax.experimental import pallas as pl
from jax.experimental.pallas import tpu as pltpu
from jax.experimental.pallas import tpu_sc as plsc

_NC = 2
_NS = 16
_NW = _NC * _NS
_CH = 100


def _elu(x):
    return jnp.where(x > 0, x, jnp.exp(x) - 1.0)


def _dotT(x, w):
    return lax.dot_general(x, w, (((1,), (1,)), ((), ())),
                           preferred_element_type=jnp.float32)


def _proj_body(n_ref, ws_ref, wr_ref, s_ref, r_ref):
    n = n_ref[...]
    s_ref[...] = _dotT(n, ws_ref[...])
    r_ref[...] = _dotT(n, wr_ref[...])


def _project_nodes(nodes2d, w1s, w1r, row_blk):
    n, d = nodes2d.shape
    return pl.pallas_call(
        _proj_body,
        grid=(n // row_blk,),
        in_specs=[pl.BlockSpec((row_blk, d), lambda i: (i, 0)),
                  pl.BlockSpec(w1s.shape, lambda i: (0, 0)),
                  pl.BlockSpec(w1r.shape, lambda i: (0, 0))],
        out_specs=[pl.BlockSpec((row_blk, d), lambda i: (i, 0)),
                   pl.BlockSpec((row_blk, d), lambda i: (i, 0))],
        out_shape=[jax.ShapeDtypeStruct((n, d), jnp.float32),
                   jax.ShapeDtypeStruct((n, d), jnp.float32)],
    )(nodes2d, w1s, w1r)


def _pair_split_body(p_ref, p0_ref, p1_ref):
    pr = p_ref[...]
    p0_ref[...] = pr[0]
    p1_ref[...] = pr[1]


def _pair_split(pair_t):
    n = pair_t.shape[1]
    return pl.pallas_call(
        _pair_split_body,
        out_shape=[jax.ShapeDtypeStruct((n,), jnp.int32),
                   jax.ShapeDtypeStruct((n,), jnp.int32)],
    )(pair_t)


def _edge_mlp_body(h_ref, et_ref, w1e_ref, b1_ref, w2_ref, b2_ref,
                   x_ref, st_ref):
    i = pl.program_id(0)
    ew = lax.dot_general(et_ref[...], w1e_ref[...], (((0,), (1,)), ((), ())),
                         preferred_element_type=jnp.float32)
    h = h_ref[...] + ew + b1_ref[...]
    h = _elu(h)
    x = _elu(_dotT(h, w2_ref[...]) + b2_ref[...])
    c = x.shape[1]
    r = x.shape[0]
    x_ref[...] = jnp.concatenate(
        [x, jnp.ones((r, 1), jnp.float32),
         jnp.zeros((r, 127 - c), jnp.float32)], axis=1)
    ps = jnp.sum(x, axis=0, keepdims=True)
    pq = jnp.sum(x * x, axis=0, keepdims=True)
    upd = jnp.concatenate([ps, pq, jnp.zeros((6, c), jnp.float32)], axis=0)

    @pl.when(i == 0)
    def _():
        st_ref[...] = upd

    @pl.when(i > 0)
    def _():
        st_ref[...] += upd


def _edge_mlp(h, e_t, w1e, b1, w2, b2, row_blk):
    n, dh = h.shape
    de = e_t.shape[0]
    c = w2.shape[0]
    return pl.pallas_call(
        _edge_mlp_body,
        grid=(n // row_blk,),
        in_specs=[pl.BlockSpec((row_blk, dh), lambda i: (i, 0)),
                  pl.BlockSpec((de, row_blk), lambda i: (0, i)),
                  pl.BlockSpec(w1e.shape, lambda i: (0, 0)),
                  pl.BlockSpec((1, dh), lambda i: (0, 0)),
                  pl.BlockSpec(w2.shape, lambda i: (0, 0)),
                  pl.BlockSpec((1, c), lambda i: (0, 0))],
        out_specs=[pl.BlockSpec((row_blk, 128), lambda i: (i, 0)),
                   pl.BlockSpec((8, c), lambda i: (0, 0))],
        out_shape=[jax.ShapeDtypeStruct((n, 128), jnp.float32),
                   jax.ShapeDtypeStruct((8, c), jnp.float32)],
    )(h, e_t, w1e, b1, w2, b2)


def _edge_fin_body(x_ref, et_ref, a_ref, c_ref, net_ref):
    dc = et_ref.shape[0]
    y = x_ref[:, :dc] * a_ref[...] + c_ref[...]
    net_ref[...] = jnp.transpose(y) + et_ref[...]


def _edge_fin(x128, e_t, a, c, row_blk):
    dc, n = e_t.shape
    return pl.pallas_call(
        _edge_fin_body,
        grid=(n // row_blk,),
        in_specs=[pl.BlockSpec((row_blk, 128), lambda i: (i, 0)),
                  pl.BlockSpec((dc, row_blk), lambda i: (0, i)),
                  pl.BlockSpec((1, dc), lambda i: (0, 0)),
                  pl.BlockSpec((1, dc), lambda i: (0, 0))],
        out_specs=pl.BlockSpec((dc, row_blk), lambda i: (0, i)),
        out_shape=jax.ShapeDtypeStruct((dc, n), jnp.float32),
    )(x128, e_t, a, c)


def _node_mlp_body(n_ref, s0_ref, s1_ref, ae_ref, ce_ref, w1a_ref, w1b_ref,
                   b1_ref, w2_ref, b2_ref, x_ref, st_ref):
    i = pl.program_id(0)
    dc = w1b_ref.shape[1]
    st = s0_ref[...] + s1_ref[...]
    nf = st[:, :dc] * ae_ref[...] + st[:, dc:dc + 1] * ce_ref[...]
    h = (_dotT(n_ref[...], w1a_ref[...]) + _dotT(nf, w1b_ref[...])
         + b1_ref[...])
    h = _elu(h)
    x = _elu(_dotT(h, w2_ref[...]) + b2_ref[...])
    x_ref[...] = x
    c = x.shape[1]
    ps = jnp.sum(x, axis=0, keepdims=True)
    pq = jnp.sum(x * x, axis=0, keepdims=True)
    upd = jnp.concatenate([ps, pq, jnp.zeros((6, c), jnp.float32)], axis=0)

    @pl.when(i == 0)
    def _():
        st_ref[...] = upd

    @pl.when(i > 0)
    def _():
        st_ref[...] += upd


def _node_mlp(n2, s0, s1, ae, ce, w1a, w1b, b1, w2, b2, row_blk):
    n, dn = n2.shape
    dc = s0.shape[1]
    dce = ae.shape[1]
    dh = w1a.shape[0]
    do = w2.shape[0]
    return pl.pallas_call(
        _node_mlp_body,
        grid=(n // row_blk,),
        in_specs=[pl.BlockSpec((row_blk, dn), lambda i: (i, 0)),
                  pl.BlockSpec((row_blk, dc), lambda i: (i, 0)),
                  pl.BlockSpec((row_blk, dc), lambda i: (i, 0)),
                  pl.BlockSpec((1, dce), lambda i: (0, 0)),
                  pl.BlockSpec((1, dce), lambda i: (0, 0)),
                  pl.BlockSpec(w1a.shape, lambda i: (0, 0)),
                  pl.BlockSpec(w1b.shape, lambda i: (0, 0)),
                  pl.BlockSpec((1, dh), lambda i: (0, 0)),
                  pl.BlockSpec(w2.shape, lambda i: (0, 0)),
                  pl.BlockSpec((1, do), lambda i: (0, 0))],
        out_specs=[pl.BlockSpec((row_blk, do), lambda i: (i, 0)),
                   pl.BlockSpec((8, do), lambda i: (0, 0))],
        out_shape=[jax.ShapeDtypeStruct((n, do), jnp.float32),
                   jax.ShapeDtypeStruct((8, do), jnp.float32)],
    )(n2, s0, s1, ae, ce, w1a, w1b, b1, w2, b2)


def _node_fin_body(x_ref, n_ref, a_ref, c_ref, o_ref):
    o_ref[...] = x_ref[...] * a_ref[...] + c_ref[...] + n_ref[...]


def _node_fin(x, n2, a, c, row_blk):
    n, dn = x.shape
    return pl.pallas_call(
        _node_fin_body,
        grid=(n // row_blk,),
        in_specs=[pl.BlockSpec((row_blk, dn), lambda i: (i, 0)),
                  pl.BlockSpec((row_blk, dn), lambda i: (i, 0)),
                  pl.BlockSpec((1, dn), lambda i: (0, 0)),
                  pl.BlockSpec((1, dn), lambda i: (0, 0))],
        out_specs=pl.BlockSpec((row_blk, dn), lambda i: (i, 0)),
        out_shape=jax.ShapeDtypeStruct((n, dn), jnp.float32),
    )(x, n2, a, c)


_CHF = 128


def _sc_gather_sum(e_total, d):
    epw = e_total // _NW
    ncf = epw // _CHF
    tail = epw - ncf * _CHF
    assert ncf % 2 == 0 and e_total % _NW == 0 and tail % 8 == 0
    mesh = plsc.VectorSubcoreMesh(core_axis_name="c", subcore_axis_name="s")

    @functools.partial(
        pl.kernel,
        out_type=jax.ShapeDtypeStruct((e_total, d), jnp.float32),
        mesh=mesh,
        scratch_types=[
            pltpu.VMEM((2, _CHF), jnp.int32),
            pltpu.VMEM((2, _CHF), jnp.int32),
            pltpu.VMEM((2, _CHF, d), jnp.float32),
            pltpu.VMEM((2, _CHF, d), jnp.float32),
            pltpu.VMEM((2, _CHF, d), jnp.float32),
            pltpu.SemaphoreType.DMA((2,)),
            pltpu.SemaphoreType.DMA((2,)),
            pltpu.SemaphoreType.DMA((2,)),
            pltpu.SemaphoreType.DMA((2,)),
        ],
    )
    def k(ts_hbm, tr_hbm, i0_hbm, i1_hbm, out_hbm, ia, ib, bufa, bufb, bufo,
          si, sa, sb, so):
        wid = lax.axis_index("s") * _NC + lax.axis_index("c")
        base = wid * epw

        def idx_issue(j, s):
            pltpu.async_copy(i0_hbm.at[pl.ds(base + j * _CHF, _CHF)],
                             ia.at[s], si.at[s])
            pltpu.async_copy(i1_hbm.at[pl.ds(base + j * _CHF, _CHF)],
                             ib.at[s], si.at[s])

        def idx_wait(j, s):
            pltpu.make_async_copy(i0_hbm.at[pl.ds(base + j * _CHF, _CHF)],
                                  ia.at[s], si.at[s]).wait()
            pltpu.make_async_copy(i1_hbm.at[pl.ds(base + j * _CHF, _CHF)],
                                  ib.at[s], si.at[s]).wait()

        def g_issue(s):
            pltpu.async_copy(ts_hbm.at[ia.at[s]], bufa.at[s], sa.at[s])
            pltpu.async_copy(tr_hbm.at[ib.at[s]], bufb.at[s], sb.at[s])

        def g_wait(s):
            pltpu.make_async_copy(ts_hbm.at[ia.at[s]], bufa.at[s],
                                  sa.at[s]).wait()
            pltpu.make_async_copy(tr_hbm.at[ib.at[s]], bufb.at[s],
                                  sb.at[s]).wait()

        def out_wait(j, s):
            pltpu.make_async_copy(bufo.at[s],
                                  out_hbm.at[pl.ds(base + j * _CHF, _CHF)],
                                  so.at[s]).wait()

        def add_rows(s, nrows):
            def row(r, cc):
                def col(q, c3):
                    sl = (s, r, pl.ds(q * 16, 16))
                    bufo[sl] = bufa[sl] + bufb[sl]
                    return c3
                return lax.fori_loop(0, d // 16, col, cc, unroll=True)
            lax.fori_loop(0, nrows, row, 0)

        idx_issue(0, 0)
        idx_issue(1, 1)
        idx_wait(0, 0)
        g_issue(0)

        def step(s, jj):
            g_wait(s)

            @pl.when(jj >= 2)
            def _():
                out_wait(jj - 2, s)

            add_rows(s, _CHF)
            pltpu.async_copy(bufo.at[s],
                             out_hbm.at[pl.ds(base + jj * _CHF, _CHF)],
                             so.at[s])

            @pl.when(jj + 1 < ncf)
            def _():
                idx_wait(jj + 1, 1 - s)
                g_issue(1 - s)

            @pl.when(jj + 2 < ncf)
            def _():
                idx_issue(jj + 2, s)

        def body(j2, carry):
            step(0, 2 * j2)
            step(1, 2 * j2 + 1)
            return carry

        lax.fori_loop(0, ncf // 2, body, 0)
        out_wait(ncf - 2, 0)
        out_wait(ncf - 1, 1)

        if tail:
            toff = base + ncf * _CHF
            pltpu.sync_copy(i0_hbm.at[pl.ds(toff, tail)],
                            ia.at[0, pl.ds(0, tail)])
            pltpu.sync_copy(i1_hbm.at[pl.ds(toff, tail)],
                            ib.at[0, pl.ds(0, tail)])
            ca = pltpu.async_copy(ts_hbm.at[ia.at[0, pl.ds(0, tail)]],
                                  bufa.at[0, pl.ds(0, tail)], sa.at[0])
            cb = pltpu.async_copy(tr_hbm.at[ib.at[0, pl.ds(0, tail)]],
                                  bufb.at[0, pl.ds(0, tail)], sb.at[0])
            ca.wait()
            cb.wait()
            add_rows(0, tail)
            pltpu.sync_copy(bufo.at[0, pl.ds(0, tail)],
                            out_hbm.at[pl.ds(toff, tail)])

    return k


def _sc_scatter_add(e_total, n_nodes, d):
    epw = e_total // _NW
    ncf = epw // _CHF
    tail = epw - ncf * _CHF
    assert ncf % 2 == 0 and e_total % _NW == 0 and tail % 8 == 0
    npad = ((n_nodes + _NS * _CHF - 1) // (_NS * _CHF)) * _NS * _CHF
    blk_per_tile = npad // (_NS * _CHF)
    mesh = plsc.VectorSubcoreMesh(core_axis_name="c", subcore_axis_name="s")

    @functools.partial(
        pl.kernel,
        out_type=jax.ShapeDtypeStruct((2, n_nodes, d), jnp.float32),
        mesh=mesh,
        scratch_types=[
            pltpu.VMEM((2, _CHF), jnp.int32),
            pltpu.VMEM((2, _CHF), jnp.int32),
            pltpu.VMEM((2, _CHF, d), jnp.float32),
            pltpu.VMEM_SHARED((npad, d), jnp.float32),
            pltpu.SemaphoreType.DMA((2,)),
            pltpu.SemaphoreType.DMA((2,)),
            pltpu.SemaphoreType.DMA((2,)),
        ],
    )
    def k(y_hbm, p0_hbm, p1_hbm, out_hbm, ia, ib, ybuf, acc,
          sl, t0, t1):
        cid = lax.axis_index("c")
        sid = lax.axis_index("s")

        def zrow(r, cc):
            def zcol(q, c3):
                ybuf[0, r, pl.ds(q * 16, 16)] = jnp.zeros((16,), jnp.float32)
                return c3
            return lax.fori_loop(0, d // 16, zcol, cc, unroll=True)
        lax.fori_loop(0, _CHF, zrow, 0)
        for kk in range(blk_per_tile):
            pltpu.sync_copy(ybuf.at[0],
                            acc.at[pl.ds((sid * blk_per_tile + kk) * _CHF,
                                         _CHF)])

        wid = sid * _NC + cid
        base = wid * epw
        plsc.subcore_barrier()

        def load(j, s):
            pltpu.async_copy(p0_hbm.at[pl.ds(base + j * _CHF, _CHF)],
                             ia.at[s], sl.at[s])
            pltpu.async_copy(p1_hbm.at[pl.ds(base + j * _CHF, _CHF)],
                             ib.at[s], sl.at[s])
            pltpu.async_copy(y_hbm.at[pl.ds(base + j * _CHF, _CHF)],
                             ybuf.at[s], sl.at[s])

        def load_wait(j, s):
            pltpu.make_async_copy(p0_hbm.at[pl.ds(base + j * _CHF, _CHF)],
                                  ia.at[s], sl.at[s]).wait()
            pltpu.make_async_copy(p1_hbm.at[pl.ds(base + j * _CHF, _CHF)],
                                  ib.at[s], sl.at[s]).wait()
            pltpu.make_async_copy(y_hbm.at[pl.ds(base + j * _CHF, _CHF)],
                                  ybuf.at[s], sl.at[s]).wait()

        load(0, 0)
        load(1, 1)

        def step(s, jj):
            load_wait(jj, s)
            c0 = pltpu.async_copy(ybuf.at[s], acc.at[ia.at[s]], t0.at[s],
                                  add=True)
            c1 = pltpu.async_copy(ybuf.at[s], acc.at[ib.at[s]], t1.at[s],
                                  add=True)
            c0.wait()
            c1.wait()

            @pl.when(jj + 2 < ncf)
            def _():
                load(jj + 2, s)

        def body(j2, carry):
            step(0, 2 * j2)
            step(1, 2 * j2 + 1)
            return carry

        lax.fori_loop(0, ncf // 2, body, 0)

        if tail:
            toff = base + ncf * _CHF
            pltpu.sync_copy(p0_hbm.at[pl.ds(toff, tail)],
                            ia.at[0, pl.ds(0, tail)])
            pltpu.sync_copy(p1_hbm.at[pl.ds(toff, tail)],
                            ib.at[0, pl.ds(0, tail)])
            pltpu.sync_copy(y_hbm.at[pl.ds(toff, tail)],
                            ybuf.at[0, pl.ds(0, tail)])
            c0 = pltpu.async_copy(ybuf.at[0, pl.ds(0, tail)],
                                  acc.at[ia.at[0, pl.ds(0, tail)]],
                                  t0.at[0], add=True)
            c1 = pltpu.async_copy(ybuf.at[0, pl.ds(0, tail)],
                                  acc.at[ib.at[0, pl.ds(0, tail)]],
                                  t1.at[0], add=True)
            c0.wait()
            c1.wait()

        plsc.subcore_barrier()

        @pl.when(sid == 0)
        def _():
            pltpu.sync_copy(acc.at[pl.ds(0, n_nodes)], out_hbm.at[cid])

    return k


def kernel(nodes, edges, pair, eW1, eb1, eW2, eb2, eg, ebt,
           nW1, nb1, nW2, nb2, ng, nbt):
    n_nodes = nodes.shape[1]
    n_edges = edges.shape[1]
    dn = nodes.shape[2]
    de = edges.shape[2]
    n2 = nodes[0]
    e_t = jnp.transpose(edges[0])
    pair_t = jnp.transpose(pair)

    w1s = eW1[:, :dn]
    w1r = eW1[:, dn:2 * dn]
    w1e = eW1[:, 2 * dn:]

    ts, tr = _project_nodes(n2, w1s, w1r, 1000)

    p0, p1 = _pair_split(pair_t)

    h = _sc_gather_sum(n_edges, dn)(ts, tr, p0, p1)

    x128, est = _edge_mlp(h, e_t, w1e, eb1.reshape(1, -1), eW2,
                          eb2.reshape(1, -1), 2560)
    mean_e = est[0] / n_edges
    var_e = est[1] / n_edges - mean_e * mean_e
    a_e = eg * lax.rsqrt(var_e + 1e-5)
    c_e = ebt - mean_e * a_e

    s_part = _sc_scatter_add(n_edges, n_nodes, 128)(x128, p0, p1)

    new_et = _edge_fin(x128, e_t, a_e.reshape(1, -1), c_e.reshape(1, -1),
                       2560)
    new_e = jnp.transpose(new_et)

    x_n, nst = _node_mlp(n2, s_part[0], s_part[1], a_e.reshape(1, -1),
                         c_e.reshape(1, -1), nW1[:, :dn],
                         nW1[:, dn:], nb1.reshape(1, -1), nW2,
                         nb2.reshape(1, -1), 1000)
    mean_n = nst[0] / n_nodes
    var_n = nst[1] / n_nodes - mean_n * mean_n
    a_n = ng * lax.rsqrt(var_n + 1e-5)
    c_n = nbt - mean_n * a_n

    new_n = _node_fin(x_n, n2, a_n.reshape(1, -1), c_n.reshape(1, -1), 1000)

    return new_n.reshape(1, n_nodes, dn), new_e.reshape(1, n_edges, de)


# speedup vs baseline: 2.9277x; 1.0939x over previous
import functools

import jax
import jax.numpy as jnp
from jax import lax
from jax.experimental import pallas as pl
from jax.experimental.pallas import tpu as pltpu
from jax.experimental.pallas import tpu_sc as plsc

_NC = 2
_NS = 16
_NW = _NC * _NS
_CH = 100


def _elu(x):
    return jnp.where(x > 0, x, jnp.exp(x) - 1.0)


def _dotT(x, w):
    return lax.dot_general(x, w, (((1,), (1,)), ((), ())),
                           preferred_element_type=jnp.float32)


def _proj_body(n_ref, ws_ref, wr_ref, s_ref, r_ref):
    n = n_ref[...]
    s_ref[...] = _dotT(n, ws_ref[...])
    r_ref[...] = _dotT(n, wr_ref[...])


def _project_nodes(nodes2d, w1s, w1r, row_blk):
    n, d = nodes2d.shape
    return pl.pallas_call(
        _proj_body,
        grid=(n // row_blk,),
        in_specs=[pl.BlockSpec((row_blk, d), lambda i: (i, 0)),
                  pl.BlockSpec(w1s.shape, lambda i: (0, 0)),
                  pl.BlockSpec(w1r.shape, lambda i: (0, 0))],
        out_specs=[pl.BlockSpec((row_blk, d), lambda i: (i, 0)),
                   pl.BlockSpec((row_blk, d), lambda i: (i, 0))],
        out_shape=[jax.ShapeDtypeStruct((n, d), jnp.float32),
                   jax.ShapeDtypeStruct((n, d), jnp.float32)],
    )(nodes2d, w1s, w1r)


def _pair_split_body(p_ref, p0_ref, p1_ref):
    pr = p_ref[...]
    p0_ref[...] = pr[0]
    p1_ref[...] = pr[1]


def _pair_split(pair_t):
    n = pair_t.shape[1]
    return pl.pallas_call(
        _pair_split_body,
        out_shape=[jax.ShapeDtypeStruct((n,), jnp.int32),
                   jax.ShapeDtypeStruct((n,), jnp.int32)],
    )(pair_t)


def _edge_mlp_body(h_ref, et_ref, w1e_ref, b1_ref, w2_ref, b2_ref,
                   x_ref, st_ref):
    i = pl.program_id(0)
    ew = lax.dot_general(et_ref[...], w1e_ref[...], (((0,), (1,)), ((), ())),
                         preferred_element_type=jnp.float32)
    h = h_ref[...] + ew + b1_ref[...]
    h = _elu(h)
    x = _elu(_dotT(h, w2_ref[...]) + b2_ref[...])
    c = x.shape[1]
    r = x.shape[0]
    x_ref[...] = jnp.concatenate(
        [x, jnp.ones((r, 1), jnp.float32),
         jnp.zeros((r, 127 - c), jnp.float32)], axis=1)
    ps = jnp.sum(x, axis=0, keepdims=True)
    pq = jnp.sum(x * x, axis=0, keepdims=True)
    upd = jnp.concatenate([ps, pq, jnp.zeros((6, c), jnp.float32)], axis=0)

    @pl.when(i == 0)
    def _():
        st_ref[...] = upd

    @pl.when(i > 0)
    def _():
        st_ref[...] += upd


def _edge_mlp(h, e_t, w1e, b1, w2, b2, row_blk, off):
    n, dh = h.shape
    de = e_t.shape[0]
    c = w2.shape[0]
    return pl.pallas_call(
        _edge_mlp_body,
        grid=(n // row_blk,),
        in_specs=[pl.BlockSpec((row_blk, dh), lambda i: (i, 0)),
                  pl.BlockSpec((de, row_blk), lambda i: (0, i + off)),
                  pl.BlockSpec(w1e.shape, lambda i: (0, 0)),
                  pl.BlockSpec((1, dh), lambda i: (0, 0)),
                  pl.BlockSpec(w2.shape, lambda i: (0, 0)),
                  pl.BlockSpec((1, c), lambda i: (0, 0))],
        out_specs=[pl.BlockSpec((row_blk, 128), lambda i: (i, 0)),
                   pl.BlockSpec((8, c), lambda i: (0, 0))],
        out_shape=[jax.ShapeDtypeStruct((n, 128), jnp.float32),
                   jax.ShapeDtypeStruct((8, c), jnp.float32)],
    )(h, e_t, w1e, b1, w2, b2)


def _edge_fin_body(x_ref, et_ref, a_ref, c_ref, net_ref):
    dc = et_ref.shape[0]
    y = x_ref[:, :dc] * a_ref[...] + c_ref[...]
    net_ref[...] = jnp.transpose(y) + et_ref[...]


def _edge_fin(x128, e_t, a, c, row_blk, off, prev=None):
    dc, n = e_t.shape
    nseg = x128.shape[0]
    if prev is None:
        return pl.pallas_call(
            _edge_fin_body,
            grid=(nseg // row_blk,),
            in_specs=[pl.BlockSpec((row_blk, 128), lambda i: (i, 0)),
                      pl.BlockSpec((dc, row_blk), lambda i: (0, i + off)),
                      pl.BlockSpec((1, dc), lambda i: (0, 0)),
                      pl.BlockSpec((1, dc), lambda i: (0, 0))],
            out_specs=pl.BlockSpec((dc, row_blk), lambda i: (0, i + off)),
            out_shape=jax.ShapeDtypeStruct((dc, n), jnp.float32),
        )(x128, e_t, a, c)
    return pl.pallas_call(
        lambda p_ref, x_ref, et_ref, a_ref, c_ref, net_ref: (
            _edge_fin_body(x_ref, et_ref, a_ref, c_ref, net_ref)),
        grid=(nseg // row_blk,),
        in_specs=[pl.BlockSpec(memory_space=pl.ANY),
                  pl.BlockSpec((row_blk, 128), lambda i: (i, 0)),
                  pl.BlockSpec((dc, row_blk), lambda i: (0, i + off)),
                  pl.BlockSpec((1, dc), lambda i: (0, 0)),
                  pl.BlockSpec((1, dc), lambda i: (0, 0))],
        out_specs=pl.BlockSpec((dc, row_blk), lambda i: (0, i + off)),
        out_shape=jax.ShapeDtypeStruct((dc, n), jnp.float32),
        input_output_aliases={0: 0},
    )(prev, x128, e_t, a, c)


def _node_mlp_body(n_ref, s0_ref, s1_ref, s2_ref, s3_ref, ae_ref, ce_ref,
                   w1a_ref, w1b_ref, b1_ref, w2_ref, b2_ref, x_ref, st_ref):
    i = pl.program_id(0)
    dc = w1b_ref.shape[1]
    st = (s0_ref[...] + s1_ref[...]) + (s2_ref[...] + s3_ref[...])
    nf = st[:, :dc] * ae_ref[...] + st[:, dc:dc + 1] * ce_ref[...]
    h = (_dotT(n_ref[...], w1a_ref[...]) + _dotT(nf, w1b_ref[...])
         + b1_ref[...])
    h = _elu(h)
    x = _elu(_dotT(h, w2_ref[...]) + b2_ref[...])
    x_ref[...] = x
    c = x.shape[1]
    ps = jnp.sum(x, axis=0, keepdims=True)
    pq = jnp.sum(x * x, axis=0, keepdims=True)
    upd = jnp.concatenate([ps, pq, jnp.zeros((6, c), jnp.float32)], axis=0)

    @pl.when(i == 0)
    def _():
        st_ref[...] = upd

    @pl.when(i > 0)
    def _():
        st_ref[...] += upd


def _node_mlp(n2, s0, s1, s2, s3, ae, ce, w1a, w1b, b1, w2, b2, row_blk):
    n, dn = n2.shape
    dc = s0.shape[1]
    dce = ae.shape[1]
    dh = w1a.shape[0]
    do = w2.shape[0]
    return pl.pallas_call(
        _node_mlp_body,
        grid=(n // row_blk,),
        in_specs=[pl.BlockSpec((row_blk, dn), lambda i: (i, 0)),
                  pl.BlockSpec((row_blk, dc), lambda i: (i, 0)),
                  pl.BlockSpec((row_blk, dc), lambda i: (i, 0)),
                  pl.BlockSpec((row_blk, dc), lambda i: (i, 0)),
                  pl.BlockSpec((row_blk, dc), lambda i: (i, 0)),
                  pl.BlockSpec((1, dce), lambda i: (0, 0)),
                  pl.BlockSpec((1, dce), lambda i: (0, 0)),
                  pl.BlockSpec(w1a.shape, lambda i: (0, 0)),
                  pl.BlockSpec(w1b.shape, lambda i: (0, 0)),
                  pl.BlockSpec((1, dh), lambda i: (0, 0)),
                  pl.BlockSpec(w2.shape, lambda i: (0, 0)),
                  pl.BlockSpec((1, do), lambda i: (0, 0))],
        out_specs=[pl.BlockSpec((row_blk, do), lambda i: (i, 0)),
                   pl.BlockSpec((8, do), lambda i: (0, 0))],
        out_shape=[jax.ShapeDtypeStruct((n, do), jnp.float32),
                   jax.ShapeDtypeStruct((8, do), jnp.float32)],
    )(n2, s0, s1, s2, s3, ae, ce, w1a, w1b, b1, w2, b2)


def _node_fin_body(x_ref, n_ref, a_ref, c_ref, o_ref):
    o_ref[...] = x_ref[...] * a_ref[...] + c_ref[...] + n_ref[...]


def _node_fin(x, n2, a, c, row_blk):
    n, dn = x.shape
    return pl.pallas_call(
        _node_fin_body,
        grid=(n // row_blk,),
        in_specs=[pl.BlockSpec((row_blk, dn), lambda i: (i, 0)),
                  pl.BlockSpec((row_blk, dn), lambda i: (i, 0)),
                  pl.BlockSpec((1, dn), lambda i: (0, 0)),
                  pl.BlockSpec((1, dn), lambda i: (0, 0))],
        out_specs=pl.BlockSpec((row_blk, dn), lambda i: (i, 0)),
        out_shape=jax.ShapeDtypeStruct((n, dn), jnp.float32),
    )(x, n2, a, c)


_CHF = 128


def _sc_gather_sum(e_total, d):
    epw = e_total // _NW
    ncf = epw // _CHF
    tail = epw - ncf * _CHF
    assert ncf % 2 == 0 and e_total % _NW == 0 and tail % 8 == 0
    mesh = plsc.VectorSubcoreMesh(core_axis_name="c", subcore_axis_name="s")

    @functools.partial(
        pl.kernel,
        out_type=jax.ShapeDtypeStruct((e_total, d), jnp.float32),
        mesh=mesh,
        scratch_types=[
            pltpu.VMEM((2, _CHF), jnp.int32),
            pltpu.VMEM((2, _CHF), jnp.int32),
            pltpu.VMEM((2, _CHF, d), jnp.float32),
            pltpu.VMEM((2, _CHF, d), jnp.float32),
            pltpu.VMEM((2, _CHF, d), jnp.float32),
            pltpu.SemaphoreType.DMA((2,)),
            pltpu.SemaphoreType.DMA((2,)),
            pltpu.SemaphoreType.DMA((2,)),
            pltpu.SemaphoreType.DMA((2,)),
        ],
    )
    def k(ts_hbm, tr_hbm, i0_hbm, i1_hbm, out_hbm, ia, ib, bufa, bufb, bufo,
          si, sa, sb, so):
        wid = lax.axis_index("s") * _NC + lax.axis_index("c")
        base = wid * epw

        def idx_issue(j, s):
            pltpu.async_copy(i0_hbm.at[pl.ds(base + j * _CHF, _CHF)],
                             ia.at[s], si.at[s])
            pltpu.async_copy(i1_hbm.at[pl.ds(base + j * _CHF, _CHF)],
                             ib.at[s], si.at[s])

        def idx_wait(j, s):
            pltpu.make_async_copy(i0_hbm.at[pl.ds(base + j * _CHF, _CHF)],
                                  ia.at[s], si.at[s]).wait()
            pltpu.make_async_copy(i1_hbm.at[pl.ds(base + j * _CHF, _CHF)],
                                  ib.at[s], si.at[s]).wait()

        def g_issue(s):
            pltpu.async_copy(ts_hbm.at[ia.at[s]], bufa.at[s], sa.at[s])
            pltpu.async_copy(tr_hbm.at[ib.at[s]], bufb.at[s], sb.at[s])

        def g_wait(s):
            pltpu.make_async_copy(ts_hbm.at[ia.at[s]], bufa.at[s],
                                  sa.at[s]).wait()
            pltpu.make_async_copy(tr_hbm.at[ib.at[s]], bufb.at[s],
                                  sb.at[s]).wait()

        def out_wait(j, s):
            pltpu.make_async_copy(bufo.at[s],
                                  out_hbm.at[pl.ds(base + j * _CHF, _CHF)],
                                  so.at[s]).wait()

        def add_rows(s, nrows):
            def row(r, cc):
                def col(q, c3):
                    sl = (s, r, pl.ds(q * 16, 16))
                    bufo[sl] = bufa[sl] + bufb[sl]
                    return c3
                return lax.fori_loop(0, d // 16, col, cc, unroll=True)
            lax.fori_loop(0, nrows, row, 0)

        idx_issue(0, 0)
        idx_issue(1, 1)
        idx_wait(0, 0)
        g_issue(0)

        def step(s, jj):
            g_wait(s)

            @pl.when(jj >= 2)
            def _():
                out_wait(jj - 2, s)

            add_rows(s, _CHF)
            pltpu.async_copy(bufo.at[s],
                             out_hbm.at[pl.ds(base + jj * _CHF, _CHF)],
                             so.at[s])

            @pl.when(jj + 1 < ncf)
            def _():
                idx_wait(jj + 1, 1 - s)
                g_issue(1 - s)

            @pl.when(jj + 2 < ncf)
            def _():
                idx_issue(jj + 2, s)

        def body(j2, carry):
            step(0, 2 * j2)
            step(1, 2 * j2 + 1)
            return carry

        lax.fori_loop(0, ncf // 2, body, 0)
        out_wait(ncf - 2, 0)
        out_wait(ncf - 1, 1)

        if tail:
            toff = base + ncf * _CHF
            pltpu.sync_copy(i0_hbm.at[pl.ds(toff, tail)],
                            ia.at[0, pl.ds(0, tail)])
            pltpu.sync_copy(i1_hbm.at[pl.ds(toff, tail)],
                            ib.at[0, pl.ds(0, tail)])
            ca = pltpu.async_copy(ts_hbm.at[ia.at[0, pl.ds(0, tail)]],
                                  bufa.at[0, pl.ds(0, tail)], sa.at[0])
            cb = pltpu.async_copy(tr_hbm.at[ib.at[0, pl.ds(0, tail)]],
                                  bufb.at[0, pl.ds(0, tail)], sb.at[0])
            ca.wait()
            cb.wait()
            add_rows(0, tail)
            pltpu.sync_copy(bufo.at[0, pl.ds(0, tail)],
                            out_hbm.at[pl.ds(toff, tail)])

    return k


def _sc_scatter_add(e_total, n_nodes, d):
    epw = e_total // _NW
    ncf = epw // _CHF
    tail = epw - ncf * _CHF
    assert ncf % 2 == 0 and e_total % _NW == 0 and tail % 8 == 0
    npad = ((n_nodes + _NS * _CHF - 1) // (_NS * _CHF)) * _NS * _CHF
    blk_per_tile = npad // (_NS * _CHF)
    mesh = plsc.VectorSubcoreMesh(core_axis_name="c", subcore_axis_name="s")

    @functools.partial(
        pl.kernel,
        out_type=jax.ShapeDtypeStruct((2, n_nodes, d), jnp.float32),
        mesh=mesh,
        scratch_types=[
            pltpu.VMEM((2, _CHF), jnp.int32),
            pltpu.VMEM((2, _CHF), jnp.int32),
            pltpu.VMEM((2, _CHF, d), jnp.float32),
            pltpu.VMEM_SHARED((npad, d), jnp.float32),
            pltpu.SemaphoreType.DMA((2,)),
            pltpu.SemaphoreType.DMA((2,)),
            pltpu.SemaphoreType.DMA((2,)),
        ],
    )
    def k(y_hbm, p0_hbm, p1_hbm, out_hbm, ia, ib, ybuf, acc,
          sl, t0, t1):
        cid = lax.axis_index("c")
        sid = lax.axis_index("s")

        def zrow(r, cc):
            def zcol(q, c3):
                ybuf[0, r, pl.ds(q * 16, 16)] = jnp.zeros((16,), jnp.float32)
                return c3
            return lax.fori_loop(0, d // 16, zcol, cc, unroll=True)
        lax.fori_loop(0, _CHF, zrow, 0)
        for kk in range(blk_per_tile):
            pltpu.sync_copy(ybuf.at[0],
                            acc.at[pl.ds((sid * blk_per_tile + kk) * _CHF,
                                         _CHF)])

        wid = sid * _NC + cid
        base = wid * epw
        plsc.subcore_barrier()

        def load(j, s):
            pltpu.async_copy(p0_hbm.at[pl.ds(base + j * _CHF, _CHF)],
                             ia.at[s], sl.at[s])
            pltpu.async_copy(p1_hbm.at[pl.ds(base + j * _CHF, _CHF)],
                             ib.at[s], sl.at[s])
            pltpu.async_copy(y_hbm.at[pl.ds(base + j * _CHF, _CHF)],
                             ybuf.at[s], sl.at[s])

        def load_wait(j, s):
            pltpu.make_async_copy(p0_hbm.at[pl.ds(base + j * _CHF, _CHF)],
                                  ia.at[s], sl.at[s]).wait()
            pltpu.make_async_copy(p1_hbm.at[pl.ds(base + j * _CHF, _CHF)],
                                  ib.at[s], sl.at[s]).wait()
            pltpu.make_async_copy(y_hbm.at[pl.ds(base + j * _CHF, _CHF)],
                                  ybuf.at[s], sl.at[s]).wait()

        load(0, 0)
        load(1, 1)

        def step(s, jj):
            load_wait(jj, s)
            c0 = pltpu.async_copy(ybuf.at[s], acc.at[ia.at[s]], t0.at[s],
                                  add=True)
            c1 = pltpu.async_copy(ybuf.at[s], acc.at[ib.at[s]], t1.at[s],
                                  add=True)
            c0.wait()
            c1.wait()

            @pl.when(jj + 2 < ncf)
            def _():
                load(jj + 2, s)

        def body(j2, carry):
            step(0, 2 * j2)
            step(1, 2 * j2 + 1)
            return carry

        lax.fori_loop(0, ncf // 2, body, 0)

        if tail:
            toff = base + ncf * _CHF
            pltpu.sync_copy(p0_hbm.at[pl.ds(toff, tail)],
                            ia.at[0, pl.ds(0, tail)])
            pltpu.sync_copy(p1_hbm.at[pl.ds(toff, tail)],
                            ib.at[0, pl.ds(0, tail)])
            pltpu.sync_copy(y_hbm.at[pl.ds(toff, tail)],
                            ybuf.at[0, pl.ds(0, tail)])
            c0 = pltpu.async_copy(ybuf.at[0, pl.ds(0, tail)],
                                  acc.at[ia.at[0, pl.ds(0, tail)]],
                                  t0.at[0], add=True)
            c1 = pltpu.async_copy(ybuf.at[0, pl.ds(0, tail)],
                                  acc.at[ib.at[0, pl.ds(0, tail)]],
                                  t1.at[0], add=True)
            c0.wait()
            c1.wait()

        plsc.subcore_barrier()

        @pl.when(sid == 0)
        def _():
            pltpu.sync_copy(acc.at[pl.ds(0, n_nodes)], out_hbm.at[cid])

    return k


def kernel(nodes, edges, pair, eW1, eb1, eW2, eb2, eg, ebt,
           nW1, nb1, nW2, nb2, ng, nbt):
    n_nodes = nodes.shape[1]
    n_edges = edges.shape[1]
    dn = nodes.shape[2]
    de = edges.shape[2]
    n2 = nodes[0]
    e_t = jnp.transpose(edges[0])
    pair_t = jnp.transpose(pair)

    w1s = eW1[:, :dn]
    w1r = eW1[:, dn:2 * dn]
    w1e = eW1[:, 2 * dn:]

    ts, tr = _project_nodes(n2, w1s, w1r, 1000)

    p0, p1 = _pair_split(pair_t)

    rb = 2560
    na = max(40960, ((n_edges // 2 + 20480) // 40960) * 40960)
    nb = n_edges - na
    p0a, p1a = p0[:na], p1[:na]
    p0b, p1b = p0[na:], p1[na:]

    ha = _sc_gather_sum(na, dn)(ts, tr, p0a, p1a)
    hb = _sc_gather_sum(nb, dn)(ts, tr, p0b, p1b)

    xa, esta = _edge_mlp(ha, e_t, w1e, eb1.reshape(1, -1), eW2,
                         eb2.reshape(1, -1), rb, 0)
    xb, estb = _edge_mlp(hb, e_t, w1e, eb1.reshape(1, -1), eW2,
                         eb2.reshape(1, -1), rb, na // rb)
    est = esta + estb
    mean_e = est[0] / n_edges
    var_e = est[1] / n_edges - mean_e * mean_e
    a_e = eg * lax.rsqrt(var_e + 1e-5)
    c_e = ebt - mean_e * a_e

    sa_ = _sc_scatter_add(na, n_nodes, 128)(xa, p0a, p1a)
    sb_ = _sc_scatter_add(nb, n_nodes, 128)(xb, p0b, p1b)

    net1 = _edge_fin(xa, e_t, a_e.reshape(1, -1), c_e.reshape(1, -1), rb, 0)
    new_et = _edge_fin(xb, e_t, a_e.reshape(1, -1), c_e.reshape(1, -1), rb,
                       na // rb, prev=net1)
    new_e = jnp.transpose(new_et)

    x_n, nst = _node_mlp(n2, sa_[0], sa_[1], sb_[0], sb_[1],
                         a_e.reshape(1, -1),
                         c_e.reshape(1, -1), nW1[:, :dn],
                         nW1[:, dn:], nb1.reshape(1, -1), nW2,
                         nb2.reshape(1, -1), 1000)
    mean_n = nst[0] / n_nodes
    var_n = nst[1] / n_nodes - mean_n * mean_n
    a_n = ng * lax.rsqrt(var_n + 1e-5)
    c_n = nbt - mean_n * a_n

    new_n = _node_fin(x_n, n2, a_n.reshape(1, -1), c_n.reshape(1, -1), 1000)

    return new_n.reshape(1, n_nodes, dn), new_e.reshape(1, n_edges, de)
